# baseline (speedup 1.0000x reference)
import jax
import jax.numpy as jnp
from jax.experimental import pallas as pl
from jax.experimental.pallas import tpu as pltpu

_TOKENS = 16384
_DIM = 2048
_EXPERTS = 64
_CHUNK = 1024
_NCHUNK = _TOKENS // _CHUNK
_NSLOT = 4


def _gate(x_hbm, w_ref, b_ref, o_ref, xbuf, sems):
    def start_copy(chunk, slot):
        pltpu.make_async_copy(
            x_hbm.at[pl.ds(chunk * _CHUNK, _CHUNK), :],
            xbuf.at[slot],
            sems.at[slot],
        ).start()

    def wait_copy(chunk, slot):
        pltpu.make_async_copy(
            x_hbm.at[pl.ds(chunk * _CHUNK, _CHUNK), :],
            xbuf.at[slot],
            sems.at[slot],
        ).wait()

    w = w_ref[...]
    b = b_ref[...]
    for s in range(_NSLOT):
        start_copy(s, s)
    for i in range(_NCHUNK):
        slot = i % _NSLOT
        wait_copy(i, slot)
        logits = jnp.dot(xbuf[slot], w, preferred_element_type=jnp.float32) + b
        m = jnp.max(logits, axis=-1, keepdims=True)
        e = jnp.exp(logits - m)
        o_ref[pl.ds(i * _CHUNK, _CHUNK), :] = e / jnp.sum(e, axis=-1, keepdims=True)
        nxt = i + _NSLOT
        if nxt < _NCHUNK:
            start_copy(nxt, slot)


def kernel(x, W, b):
    b2 = b.reshape(1, _EXPERTS)
    return pl.pallas_call(
        _gate,
        in_specs=[
            pl.BlockSpec(memory_space=pltpu.HBM),
            pl.BlockSpec(memory_space=pltpu.VMEM),
            pl.BlockSpec(memory_space=pltpu.VMEM),
        ],
        out_specs=pl.BlockSpec(memory_space=pltpu.VMEM),
        out_shape=jax.ShapeDtypeStruct((_TOKENS, _EXPERTS), jnp.float32),
        scratch_shapes=[
            pltpu.VMEM((_NSLOT, _CHUNK, _DIM), jnp.float32),
            pltpu.SemaphoreType.DMA((_NSLOT,)),
        ],
    )(x, W, b2)


# speedup vs baseline: 1.3407x; 1.3407x over previous
import jax
import jax.numpy as jnp
from jax import lax
from jax.experimental import pallas as pl
from jax.experimental.pallas import tpu as pltpu

_TOKENS = 16384
_DIM = 2048
_EXPERTS = 64
_BLK = 1024


def _gate_block(x_ref, wt_ref, b_ref, o_ref):
    logits = lax.dot_general(
        wt_ref[...], x_ref[...],
        (((1,), (1,)), ((), ())),
        preferred_element_type=jnp.float32,
    )
    logits = logits + b_ref[...].T
    m = jnp.max(logits, axis=0, keepdims=True)
    e = jnp.exp(logits - m)
    o_ref[...] = e / jnp.sum(e, axis=0, keepdims=True)


def kernel(x, W, b):
    wt = W.T
    b2 = b.reshape(1, _EXPERTS)
    grid = (_TOKENS // _BLK,)
    out = pl.pallas_call(
        _gate_block,
        grid=grid,
        in_specs=[
            pl.BlockSpec((_BLK, _DIM), lambda i: (i, 0)),
            pl.BlockSpec((_EXPERTS, _DIM), lambda i: (0, 0)),
            pl.BlockSpec((1, _EXPERTS), lambda i: (0, 0)),
        ],
        out_specs=pl.BlockSpec((_EXPERTS, _BLK), lambda i: (0, i)),
        out_shape=jax.ShapeDtypeStruct((_EXPERTS, _TOKENS), jnp.float32),
        compiler_params=pltpu.CompilerParams(
            dimension_semantics=("arbitrary",),
        ),
    )(x, wt, b2)
    return out.T
